# Initial kernel scaffold; baseline (speedup 1.0000x reference)
import functools

import jax
import jax.numpy as jnp
from jax import lax
from jax.experimental import pallas as pl
from jax.experimental.pallas import tpu as pltpu
from jax.experimental.pallas import tpu_sc as plsc

N = 10000
E = 160000
D = 256
H = 128
NT = 16
NP = 10240
STRIDE = NP // NT
EPT = E // NT
CH = 128
NCH = 80
EPTP = NCH * CH
EP = NT * EPTP

_mesh = plsc.VectorSubcoreMesh(core_axis_name="c", subcore_axis_name="s")


@functools.partial(
    pl.kernel,
    out_type=jax.ShapeDtypeStruct((2 * NP,), jnp.float32),
    mesh=_mesh,
    scratch_types=[
        pltpu.VMEM((NP,), jnp.float32),
        pltpu.VMEM((EPT,), jnp.int32),
        pltpu.VMEM((STRIDE,), jnp.float32),
        pltpu.VMEM((STRIDE,), jnp.float32),
        pltpu.VMEM_SHARED((NT * NP,), jnp.float32),
    ],
)
def _deg_kernel(idx_hbm, deg_hbm, hist_v, idx_v, acc_v, tmp_v, shared):
    c = lax.axis_index("c")
    s = lax.axis_index("s")
    pltpu.sync_copy(idx_hbm.at[pl.ds((c * NT + s) * EPT, EPT)], idx_v)

    @pl.loop(0, NP, step=16)
    def _zero(i):
        hist_v[pl.ds(i, 16)] = jnp.zeros((16,), jnp.float32)

    @pl.loop(0, EPT)
    def _count(e):
        n = idx_v[e]
        hist_v[n] = hist_v[n] + 1.0

    pltpu.sync_copy(hist_v, shared.at[pl.ds(s * NP, NP)])
    plsc.subcore_barrier()

    base = s * STRIDE

    @pl.loop(0, STRIDE, step=16)
    def _zacc(i):
        acc_v[pl.ds(i, 16)] = jnp.zeros((16,), jnp.float32)

    @pl.loop(0, NT)
    def _sum(t):
        pltpu.sync_copy(shared.at[pl.ds(t * NP + base, STRIDE)], tmp_v)

        @pl.loop(0, STRIDE, step=16)
        def _add(i):
            acc_v[pl.ds(i, 16)] = acc_v[pl.ds(i, 16)] + tmp_v[pl.ds(i, 16)]

    pltpu.sync_copy(acc_v, deg_hbm.at[pl.ds(c * NP + base, STRIDE)])


@functools.partial(
    pl.kernel,
    out_type=[
        jax.ShapeDtypeStruct((NP, H), jnp.float32),
        jax.ShapeDtypeStruct((NP, H), jnp.float32),
    ],
    mesh=_mesh,
    scratch_types=[
        pltpu.VMEM((NCH, CH), jnp.int32),
        pltpu.VMEM((NCH, CH), jnp.int32),
        pltpu.VMEM((NCH, CH), jnp.float32),
        pltpu.VMEM((CH, H), jnp.float32),
        pltpu.VMEM_SHARED((NP, H), jnp.float32),
        pltpu.SemaphoreType.DMA,
    ],
)
def _spmm_kernel(xa_hbm, xb_hbm, src_hbm, dst_hbm, w_hbm, zero_hbm,
                 outa_hbm, outb_hbm,
                 src_v, dst_v, w_v, rows_v, acc, sem):
    c = lax.axis_index("c")
    s = lax.axis_index("s")
    pltpu.sync_copy(src_hbm.at[s], src_v)
    pltpu.sync_copy(dst_hbm.at[s], dst_v)
    pltpu.sync_copy(w_hbm.at[s], w_v)
    pltpu.sync_copy(zero_hbm.at[pl.ds(s * STRIDE, STRIDE)],
                    acc.at[pl.ds(s * STRIDE, STRIDE)])
    plsc.subcore_barrier()

    @pl.loop(0, NCH)
    def _chunk(j):
        @pl.when(c == 0)
        def _():
            pltpu.sync_copy(xa_hbm.at[src_v.at[j]], rows_v)

        @pl.when(c == 1)
        def _():
            pltpu.sync_copy(xb_hbm.at[src_v.at[j]], rows_v)

        @pl.loop(0, CH)
        def _scale(r):
            wr = w_v[j, r]
            for k in range(H // 16):
                sl = (r, pl.ds(k * 16, 16))
                rows_v[sl] = rows_v[sl] * wr

        pltpu.sync_copy(rows_v, acc.at[dst_v.at[j]], add=True)

    plsc.subcore_barrier()

    @pl.when(c == 0)
    def _():
        pltpu.sync_copy(acc.at[pl.ds(s * STRIDE, STRIDE)],
                        outa_hbm.at[pl.ds(s * STRIDE, STRIDE)])

    @pl.when(c == 1)
    def _():
        pltpu.sync_copy(acc.at[pl.ds(s * STRIDE, STRIDE)],
                        outb_hbm.at[pl.ds(s * STRIDE, STRIDE)])


BN = 1280


def _mm1_body(x_ref, d_ref, w_ref, oa_ref, ob_ref):
    ns = lax.rsqrt(jnp.maximum(d_ref[0, :], 1.0)).reshape(BN, 1)
    xs = x_ref[...] * ns
    w = w_ref[...]
    oa_ref[...] = jnp.dot(xs, w[:, :H], preferred_element_type=jnp.float32)
    ob_ref[...] = jnp.dot(xs, w[:, H:], preferred_element_type=jnp.float32)


def _mm2_body(aa_ref, ab_ref, d_ref, b_ref, w_ref, oa_ref, ob_ref):
    nd = lax.rsqrt(jnp.maximum(d_ref[1, :], 1.0)).reshape(BN, 1)
    ns = lax.rsqrt(jnp.maximum(d_ref[0, :], 1.0)).reshape(BN, 1)
    b = b_ref[...]
    ha = jnp.maximum(aa_ref[...] * nd + b[0, :H], 0.0) * ns
    hb = jnp.maximum(ab_ref[...] * nd + b[0, H:], 0.0) * ns
    w = w_ref[...]
    oa_ref[...] = (jnp.dot(ha, w[:H, :H], preferred_element_type=jnp.float32)
                   + jnp.dot(hb, w[H:, :H], preferred_element_type=jnp.float32))
    ob_ref[...] = (jnp.dot(ha, w[:H, H:], preferred_element_type=jnp.float32)
                   + jnp.dot(hb, w[H:, H:], preferred_element_type=jnp.float32))


def _fin_body(aa_ref, ab_ref, d_ref, b_ref, o_ref):
    nd = lax.rsqrt(jnp.maximum(d_ref[1, :], 1.0)).reshape(BN, 1)
    b = b_ref[...]
    o_ref[:, :H] = aa_ref[...] * nd + b[0, :H]
    o_ref[:, H:] = ab_ref[...] * nd + b[0, H:]


_row_spec = pl.BlockSpec((BN, D), lambda i: (i, 0))
_half_spec = pl.BlockSpec((BN, H), lambda i: (i, 0))
_deg_spec = pl.BlockSpec((2, BN), lambda i: (0, i))
_w_spec = pl.BlockSpec((D, D), lambda i: (0, 0))
_b_spec = pl.BlockSpec((1, D), lambda i: (0, 0))

_mm1 = pl.pallas_call(
    _mm1_body,
    grid=(NP // BN,),
    in_specs=[_row_spec, _deg_spec, _w_spec],
    out_specs=[_half_spec, _half_spec],
    out_shape=[jax.ShapeDtypeStruct((NP, H), jnp.float32)] * 2,
)

_mm2 = pl.pallas_call(
    _mm2_body,
    grid=(NP // BN,),
    in_specs=[_half_spec, _half_spec, _deg_spec, _b_spec, _w_spec],
    out_specs=[_half_spec, _half_spec],
    out_shape=[jax.ShapeDtypeStruct((NP, H), jnp.float32)] * 2,
)

_fin = pl.pallas_call(
    _fin_body,
    grid=(NP // BN,),
    in_specs=[_half_spec, _half_spec, _deg_spec, _b_spec],
    out_specs=_row_spec,
    out_shape=jax.ShapeDtypeStruct((NP, D), jnp.float32),
)


def kernel(features, edge_index, edge_weight, W1, b1, W2, b2):
    src = edge_index[0]
    dst = edge_index[1]

    idx_flat = jnp.concatenate([src, dst])
    pad = EP - E
    pad_idx = (jnp.arange(pad, dtype=jnp.int32) * 37) % N
    srcp = jnp.concatenate([src, pad_idx]).reshape(NT, NCH, CH)
    dstp = jnp.concatenate([dst, pad_idx]).reshape(NT, NCH, CH)
    wp = jnp.concatenate(
        [edge_weight, jnp.zeros((pad,), jnp.float32)]).reshape(NT, NCH, CH)
    xpad = jnp.pad(features, ((0, NP - N), (0, 0)))
    zeros_half = jnp.zeros((NP, H), jnp.float32)
    b1r = b1.reshape(1, D)
    b2r = b2.reshape(1, D)

    deg = _deg_kernel(idx_flat)
    deg2 = deg.reshape(2, NP)
    x1a, x1b = _mm1(xpad, deg2, W1)
    a1a, a1b = _spmm_kernel(x1a, x1b, srcp, dstp, wp, zeros_half)
    x2a, x2b = _mm2(a1a, a1b, deg2, b1r, W2)
    a2a, a2b = _spmm_kernel(x2a, x2b, srcp, dstp, wp, zeros_half)
    out = _fin(a2a, a2b, deg2, b2r)
    return out[:N]


# speedup vs baseline: 4.4260x; 4.4260x over previous
import functools

import jax
import jax.numpy as jnp
from jax import lax
from jax.experimental import pallas as pl
from jax.experimental.pallas import tpu as pltpu
from jax.experimental.pallas import tpu_sc as plsc

N = 10000
E = 160000
D = 256
H = 128
NT = 16
NP = 10240
STRIDE = NP // NT
EPT = E // NT
CH = 128
NCH = 80
EPTP = NCH * CH
EP = NT * EPTP

_mesh = plsc.VectorSubcoreMesh(core_axis_name="c", subcore_axis_name="s")

_sc_params = pltpu.CompilerParams(needs_layout_passes=False)


@functools.partial(
    pl.kernel,
    out_type=jax.ShapeDtypeStruct((2 * NP,), jnp.float32),
    mesh=_mesh,
    scratch_types=[
        pltpu.VMEM((NP,), jnp.float32),
        pltpu.VMEM((EPT,), jnp.int32),
        pltpu.VMEM((STRIDE,), jnp.float32),
        pltpu.VMEM((STRIDE,), jnp.float32),
        pltpu.VMEM_SHARED((NT * NP,), jnp.float32),
    ],
    compiler_params=_sc_params,
)
def _deg_kernel(idx_hbm, deg_hbm, hist_v, idx_v, acc_v, tmp_v, shared):
    c = lax.axis_index("c")
    s = lax.axis_index("s")
    pltpu.sync_copy(idx_hbm.at[pl.ds((c * NT + s) * EPT, EPT)], idx_v)

    @pl.loop(0, NP, step=16)
    def _zero(i):
        hist_v[pl.ds(i, 16)] = jnp.zeros((16,), jnp.float32)

    @pl.loop(0, EPT, step=16)
    def _count(e0):
        idx16 = idx_v[pl.ds(e0, 16)]
        cnt, last = plsc.scan_count(idx16)
        plsc.addupdate_scatter(hist_v, [idx16], cnt.astype(jnp.float32),
                               mask=last)

    pltpu.sync_copy(hist_v, shared.at[pl.ds(s * NP, NP)])
    plsc.subcore_barrier()

    base = s * STRIDE

    @pl.loop(0, STRIDE, step=16)
    def _zacc(i):
        acc_v[pl.ds(i, 16)] = jnp.zeros((16,), jnp.float32)

    @pl.loop(0, NT)
    def _sum(t):
        pltpu.sync_copy(shared.at[pl.ds(t * NP + base, STRIDE)], tmp_v)

        @pl.loop(0, STRIDE, step=16)
        def _add(i):
            acc_v[pl.ds(i, 16)] = acc_v[pl.ds(i, 16)] + tmp_v[pl.ds(i, 16)]

    pltpu.sync_copy(acc_v, deg_hbm.at[pl.ds(c * NP + base, STRIDE)])


@functools.partial(
    pl.kernel,
    out_type=[
        jax.ShapeDtypeStruct((NP, H), jnp.float32),
        jax.ShapeDtypeStruct((NP, H), jnp.float32),
    ],
    mesh=_mesh,
    scratch_types=[
        pltpu.VMEM((NCH, CH), jnp.int32),
        pltpu.VMEM((NCH, CH), jnp.int32),
        pltpu.VMEM((NCH, CH), jnp.float32),
        pltpu.VMEM((CH, H), jnp.float32),
        pltpu.VMEM_SHARED((NP, H), jnp.float32),
        pltpu.SemaphoreType.DMA,
    ],
)
def _spmm_kernel(xa_hbm, xb_hbm, src_hbm, dst_hbm, w_hbm, zero_hbm,
                 outa_hbm, outb_hbm,
                 src_v, dst_v, w_v, rows_v, acc, sem):
    c = lax.axis_index("c")
    s = lax.axis_index("s")
    pltpu.sync_copy(src_hbm.at[s], src_v)
    pltpu.sync_copy(dst_hbm.at[s], dst_v)
    pltpu.sync_copy(w_hbm.at[s], w_v)
    pltpu.sync_copy(zero_hbm.at[pl.ds(s * STRIDE, STRIDE)],
                    acc.at[pl.ds(s * STRIDE, STRIDE)])
    plsc.subcore_barrier()

    @pl.loop(0, NCH)
    def _chunk(j):
        @pl.when(c == 0)
        def _():
            pltpu.sync_copy(xa_hbm.at[src_v.at[j]], rows_v)

        @pl.when(c == 1)
        def _():
            pltpu.sync_copy(xb_hbm.at[src_v.at[j]], rows_v)

        @pl.loop(0, CH, step=16)
        def _scale(r0):
            wv = w_v[j, pl.ds(r0, 16)]
            for r in range(16):
                wr = wv[r]
                for k in range(H // 16):
                    sl = (r0 + r, pl.ds(k * 16, 16))
                    rows_v[sl] = rows_v[sl] * wr

        pltpu.sync_copy(rows_v, acc.at[dst_v.at[j]], add=True)

    plsc.subcore_barrier()

    @pl.when(c == 0)
    def _():
        pltpu.sync_copy(acc.at[pl.ds(s * STRIDE, STRIDE)],
                        outa_hbm.at[pl.ds(s * STRIDE, STRIDE)])

    @pl.when(c == 1)
    def _():
        pltpu.sync_copy(acc.at[pl.ds(s * STRIDE, STRIDE)],
                        outb_hbm.at[pl.ds(s * STRIDE, STRIDE)])


BN = 1280


def _mm1_body(x_ref, d_ref, w_ref, oa_ref, ob_ref):
    ns = lax.rsqrt(jnp.maximum(d_ref[0, :], 1.0)).reshape(BN, 1)
    xs = x_ref[...] * ns
    w = w_ref[...]
    oa_ref[...] = jnp.dot(xs, w[:, :H], preferred_element_type=jnp.float32)
    ob_ref[...] = jnp.dot(xs, w[:, H:], preferred_element_type=jnp.float32)


def _mm2_body(aa_ref, ab_ref, d_ref, b_ref, w_ref, oa_ref, ob_ref):
    nd = lax.rsqrt(jnp.maximum(d_ref[1, :], 1.0)).reshape(BN, 1)
    ns = lax.rsqrt(jnp.maximum(d_ref[0, :], 1.0)).reshape(BN, 1)
    b = b_ref[...]
    ha = jnp.maximum(aa_ref[...] * nd + b[0, :H], 0.0) * ns
    hb = jnp.maximum(ab_ref[...] * nd + b[0, H:], 0.0) * ns
    w = w_ref[...]
    oa_ref[...] = (jnp.dot(ha, w[:H, :H], preferred_element_type=jnp.float32)
                   + jnp.dot(hb, w[H:, :H], preferred_element_type=jnp.float32))
    ob_ref[...] = (jnp.dot(ha, w[:H, H:], preferred_element_type=jnp.float32)
                   + jnp.dot(hb, w[H:, H:], preferred_element_type=jnp.float32))


def _fin_body(aa_ref, ab_ref, d_ref, b_ref, o_ref):
    nd = lax.rsqrt(jnp.maximum(d_ref[1, :], 1.0)).reshape(BN, 1)
    b = b_ref[...]
    o_ref[:, :H] = aa_ref[...] * nd + b[0, :H]
    o_ref[:, H:] = ab_ref[...] * nd + b[0, H:]


_row_spec = pl.BlockSpec((BN, D), lambda i: (i, 0))
_half_spec = pl.BlockSpec((BN, H), lambda i: (i, 0))
_deg_spec = pl.BlockSpec((2, BN), lambda i: (0, i))
_w_spec = pl.BlockSpec((D, D), lambda i: (0, 0))
_b_spec = pl.BlockSpec((1, D), lambda i: (0, 0))

_mm1 = pl.pallas_call(
    _mm1_body,
    grid=(NP // BN,),
    in_specs=[_row_spec, _deg_spec, _w_spec],
    out_specs=[_half_spec, _half_spec],
    out_shape=[jax.ShapeDtypeStruct((NP, H), jnp.float32)] * 2,
)

_mm2 = pl.pallas_call(
    _mm2_body,
    grid=(NP // BN,),
    in_specs=[_half_spec, _half_spec, _deg_spec, _b_spec, _w_spec],
    out_specs=[_half_spec, _half_spec],
    out_shape=[jax.ShapeDtypeStruct((NP, H), jnp.float32)] * 2,
)

_fin = pl.pallas_call(
    _fin_body,
    grid=(NP // BN,),
    in_specs=[_half_spec, _half_spec, _deg_spec, _b_spec],
    out_specs=_row_spec,
    out_shape=jax.ShapeDtypeStruct((NP, D), jnp.float32),
)


def kernel(features, edge_index, edge_weight, W1, b1, W2, b2):
    src = edge_index[0]
    dst = edge_index[1]

    idx_flat = jnp.concatenate([src, dst])
    pad = EP - E
    pad_idx = (jnp.arange(pad, dtype=jnp.int32) * 37) % N
    srcp = jnp.concatenate([src, pad_idx]).reshape(NT, NCH, CH)
    dstp = jnp.concatenate([dst, pad_idx]).reshape(NT, NCH, CH)
    wp = jnp.concatenate(
        [edge_weight, jnp.zeros((pad,), jnp.float32)]).reshape(NT, NCH, CH)
    xpad = jnp.pad(features, ((0, NP - N), (0, 0)))
    zeros_half = jnp.zeros((NP, H), jnp.float32)
    b1r = b1.reshape(1, D)
    b2r = b2.reshape(1, D)

    deg = _deg_kernel(idx_flat)
    deg2 = deg.reshape(2, NP)
    x1a, x1b = _mm1(xpad, deg2, W1)
    a1a, a1b = _spmm_kernel(x1a, x1b, srcp, dstp, wp, zeros_half)
    x2a, x2b = _mm2(a1a, a1b, deg2, b1r, W2)
    a2a, a2b = _spmm_kernel(x2a, x2b, srcp, dstp, wp, zeros_half)
    out = _fin(a2a, a2b, deg2, b2r)
    return out[:N]


# speedup vs baseline: 6.8476x; 1.5471x over previous
import functools

import jax
import jax.numpy as jnp
from jax import lax
from jax.experimental import pallas as pl
from jax.experimental.pallas import tpu as pltpu
from jax.experimental.pallas import tpu_sc as plsc

N = 10000
E = 160000
D = 256
H = 128
NT = 16
NP = 10240
STRIDE = NP // NT
EPT = E // NT
CH = 64
NCH = 162
EP = NT * NCH * CH
NBUF = 3

_mesh = plsc.VectorSubcoreMesh(core_axis_name="c", subcore_axis_name="s")

_sc_params = pltpu.CompilerParams(needs_layout_passes=False)


@functools.partial(
    pl.kernel,
    out_type=jax.ShapeDtypeStruct((2 * NP,), jnp.float32),
    mesh=_mesh,
    scratch_types=[
        pltpu.VMEM((NP,), jnp.float32),
        pltpu.VMEM((EPT,), jnp.int32),
        pltpu.VMEM((STRIDE,), jnp.float32),
        pltpu.VMEM((STRIDE,), jnp.float32),
        pltpu.VMEM_SHARED((NT * NP,), jnp.float32),
    ],
    compiler_params=_sc_params,
)
def _deg_kernel(idx_hbm, deg_hbm, hist_v, idx_v, acc_v, tmp_v, shared):
    c = lax.axis_index("c")
    s = lax.axis_index("s")
    pltpu.sync_copy(idx_hbm.at[pl.ds((c * NT + s) * EPT, EPT)], idx_v)

    @pl.loop(0, NP, step=16)
    def _zero(i):
        hist_v[pl.ds(i, 16)] = jnp.zeros((16,), jnp.float32)

    @pl.loop(0, EPT, step=16)
    def _count(e0):
        idx16 = idx_v[pl.ds(e0, 16)]
        cnt, last = plsc.scan_count(idx16)
        plsc.addupdate_scatter(hist_v, [idx16], cnt.astype(jnp.float32),
                               mask=last)

    pltpu.sync_copy(hist_v, shared.at[pl.ds(s * NP, NP)])
    plsc.subcore_barrier()

    base = s * STRIDE

    @pl.loop(0, STRIDE, step=16)
    def _zacc(i):
        acc_v[pl.ds(i, 16)] = jnp.zeros((16,), jnp.float32)

    @pl.loop(0, NT)
    def _sum(t):
        pltpu.sync_copy(shared.at[pl.ds(t * NP + base, STRIDE)], tmp_v)

        @pl.loop(0, STRIDE, step=16)
        def _add(i):
            acc_v[pl.ds(i, 16)] = acc_v[pl.ds(i, 16)] + tmp_v[pl.ds(i, 16)]

    pltpu.sync_copy(acc_v, deg_hbm.at[pl.ds(c * NP + base, STRIDE)])


@functools.partial(
    pl.kernel,
    out_type=jax.ShapeDtypeStruct((2 * NP, H), jnp.float32),
    mesh=_mesh,
    scratch_types=[
        pltpu.VMEM((NCH, CH), jnp.int32),
        pltpu.VMEM((NBUF, CH), jnp.int32),
        pltpu.VMEM((NBUF, CH), jnp.float32),
        [pltpu.VMEM((CH, H), jnp.float32)] * NBUF,
        pltpu.VMEM_SHARED((NP, H), jnp.float32),
        [pltpu.SemaphoreType.DMA] * NBUF,
        [pltpu.SemaphoreType.DMA] * NBUF,
    ],
    compiler_params=_sc_params,
)
def _spmm_kernel(x_hbm, src_hbm, dst_hbm, w_hbm, zero_hbm, out_hbm,
                 src_v, dst_v, w_v, rows, acc, gsem, ssem):
    c = lax.axis_index("c")
    s = lax.axis_index("s")
    pltpu.sync_copy(src_hbm.at[c * NT + s], src_v)

    def gstart(b, q):
        pltpu.async_copy(x_hbm.at[src_v.at[q]], rows[b], gsem[b])
        off = (s * NCH + q) * CH
        pltpu.async_copy(dst_hbm.at[pl.ds(off, CH)], dst_v.at[b], gsem[b])
        pltpu.async_copy(w_hbm.at[pl.ds(off, CH)], w_v.at[b], gsem[b])

    def gwait(b, q):
        pltpu.make_async_copy(x_hbm.at[src_v.at[q]], rows[b], gsem[b]).wait()
        off = (s * NCH + q) * CH
        pltpu.make_async_copy(dst_hbm.at[pl.ds(off, CH)], dst_v.at[b],
                              gsem[b]).wait()
        pltpu.make_async_copy(w_hbm.at[pl.ds(off, CH)], w_v.at[b],
                              gsem[b]).wait()

    def sstart(b):
        pltpu.async_copy(rows[b], acc.at[dst_v.at[b]], ssem[b], add=True)

    def swait(b):
        pltpu.make_async_copy(rows[b], acc.at[dst_v.at[b]], ssem[b]).wait()

    for b in range(NBUF - 1):
        gstart(b, b)

    pltpu.sync_copy(zero_hbm.at[pl.ds(s * STRIDE, STRIDE)],
                    acc.at[pl.ds(s * STRIDE, STRIDE)])
    plsc.subcore_barrier()

    @pl.loop(0, NCH, step=NBUF)
    def _iter(j):
        for k in range(NBUF):
            q = j + k
            b = k
            gwait(b, q)

            @pl.loop(0, CH, step=16)
            def _scale(r0):
                wv = w_v[b, pl.ds(r0, 16)]
                for r in range(16):
                    wr = wv[r]
                    for kk in range(H // 16):
                        sl = (r0 + r, pl.ds(kk * 16, 16))
                        rows[b][sl] = rows[b][sl] * wr

            sstart(b)
            nb = (k + NBUF - 1) % NBUF
            if k == 0:
                @pl.when(j > 0)
                def _():
                    swait(nb)

                gstart(nb, q + NBUF - 1)
            else:
                @pl.when(j < NCH - (NBUF - 1) - k)
                def _():
                    swait(nb)
                    gstart(nb, q + NBUF - 1)

    for b in range(NBUF):
        swait(b)
    plsc.subcore_barrier()

    pltpu.sync_copy(acc.at[pl.ds(s * STRIDE, STRIDE)],
                    out_hbm.at[pl.ds(c * NP + s * STRIDE, STRIDE)])


BN = 1280


def _mm1_body(x_ref, d_ref, w_ref, o_ref):
    ns = lax.rsqrt(jnp.maximum(d_ref[0, :], 1.0)).reshape(BN, 1)
    xs = x_ref[...] * ns
    w = w_ref[...]
    o_ref[0] = jnp.dot(xs, w[:, :H], preferred_element_type=jnp.float32)
    o_ref[1] = jnp.dot(xs, w[:, H:], preferred_element_type=jnp.float32)


def _mm2_body(a_ref, d_ref, b_ref, w_ref, o_ref):
    nd = lax.rsqrt(jnp.maximum(d_ref[1, :], 1.0)).reshape(BN, 1)
    ns = lax.rsqrt(jnp.maximum(d_ref[0, :], 1.0)).reshape(BN, 1)
    b = b_ref[...]
    ha = jnp.maximum(a_ref[0] * nd + b[0, :H], 0.0) * ns
    hb = jnp.maximum(a_ref[1] * nd + b[0, H:], 0.0) * ns
    w = w_ref[...]
    o_ref[0] = (jnp.dot(ha, w[:H, :H], preferred_element_type=jnp.float32)
                + jnp.dot(hb, w[H:, :H], preferred_element_type=jnp.float32))
    o_ref[1] = (jnp.dot(ha, w[:H, H:], preferred_element_type=jnp.float32)
                + jnp.dot(hb, w[H:, H:], preferred_element_type=jnp.float32))


def _fin_body(a_ref, d_ref, b_ref, o_ref):
    nd = lax.rsqrt(jnp.maximum(d_ref[1, :], 1.0)).reshape(BN, 1)
    b = b_ref[...]
    o_ref[:, :H] = a_ref[0] * nd + b[0, :H]
    o_ref[:, H:] = a_ref[1] * nd + b[0, H:]


_row_spec = pl.BlockSpec((BN, D), lambda i: (i, 0))
_halves_spec = pl.BlockSpec((2, BN, H), lambda i: (0, i, 0))
_deg_spec = pl.BlockSpec((2, BN), lambda i: (0, i))
_w_spec = pl.BlockSpec((D, D), lambda i: (0, 0))
_b_spec = pl.BlockSpec((1, D), lambda i: (0, 0))

_mm1 = pl.pallas_call(
    _mm1_body,
    grid=(NP // BN,),
    in_specs=[_row_spec, _deg_spec, _w_spec],
    out_specs=_halves_spec,
    out_shape=jax.ShapeDtypeStruct((2, NP, H), jnp.float32),
)

_mm2 = pl.pallas_call(
    _mm2_body,
    grid=(NP // BN,),
    in_specs=[_halves_spec, _deg_spec, _b_spec, _w_spec],
    out_specs=_halves_spec,
    out_shape=jax.ShapeDtypeStruct((2, NP, H), jnp.float32),
)

_fin = pl.pallas_call(
    _fin_body,
    grid=(NP // BN,),
    in_specs=[_halves_spec, _deg_spec, _b_spec],
    out_specs=_row_spec,
    out_shape=jax.ShapeDtypeStruct((NP, D), jnp.float32),
)


def kernel(features, edge_index, edge_weight, W1, b1, W2, b2):
    src = edge_index[0]
    dst = edge_index[1]

    idx_flat = jnp.concatenate([src, dst])
    pad = EP - E
    pad_idx = (jnp.arange(pad, dtype=jnp.int32) * 37) % N
    srcp = jnp.concatenate([src, pad_idx]).reshape(NT, NCH, CH)
    srcp2 = jnp.concatenate([srcp, srcp + NP], axis=0)
    dstp = jnp.concatenate([dst, pad_idx])
    wp = jnp.concatenate([edge_weight, jnp.zeros((pad,), jnp.float32)])
    xpad = jnp.pad(features, ((0, NP - N), (0, 0)))
    zeros_half = jnp.zeros((NP, H), jnp.float32)
    b1r = b1.reshape(1, D)
    b2r = b2.reshape(1, D)

    deg = _deg_kernel(idx_flat)
    deg2 = deg.reshape(2, NP)
    x1 = _mm1(xpad, deg2, W1)
    a1 = _spmm_kernel(x1.reshape(2 * NP, H), srcp2, dstp, wp, zeros_half)
    x2 = _mm2(a1.reshape(2, NP, H), deg2, b1r, W2)
    a2 = _spmm_kernel(x2.reshape(2 * NP, H), srcp2, dstp, wp, zeros_half)
    out = _fin(a2.reshape(2, NP, H), deg2, b2r)
    return out[:N]


# speedup vs baseline: 6.9626x; 1.0168x over previous
import functools

import jax
import jax.numpy as jnp
from jax import lax
from jax.experimental import pallas as pl
from jax.experimental.pallas import tpu as pltpu
from jax.experimental.pallas import tpu_sc as plsc

N = 10000
E = 160000
D = 256
H = 128
NT = 16
NP = 10240
STRIDE = NP // NT
EPT = E // NT
CH = 64
NCH = 162
ECT = NCH * CH
EP = NT * ECT
NBUF = 3

_mesh = plsc.VectorSubcoreMesh(core_axis_name="c", subcore_axis_name="s")

_sc_params = pltpu.CompilerParams(needs_layout_passes=False)


def _rsqrt16(x):
    x = jnp.maximum(x, 1.0)
    i = plsc.bitcast(x, jnp.int32)
    i = 0x5F3759DF - lax.shift_right_logical(i, 1)
    y = plsc.bitcast(i, jnp.float32)
    for _ in range(3):
        y = y * (1.5 - 0.5 * x * y * y)
    return y


@functools.partial(
    pl.kernel,
    out_type=[
        jax.ShapeDtypeStruct((NP,), jnp.float32),
        jax.ShapeDtypeStruct((EP,), jnp.float32),
    ],
    mesh=_mesh,
    scratch_types=[
        pltpu.VMEM((NP,), jnp.float32),
        pltpu.VMEM((ECT,), jnp.int32),
        pltpu.VMEM((ECT,), jnp.float32),
        pltpu.VMEM((STRIDE,), jnp.float32),
        pltpu.VMEM((STRIDE,), jnp.float32),
        pltpu.VMEM_SHARED((NT * NP,), jnp.float32),
    ],
    compiler_params=_sc_params,
)
def _prep_kernel(idx_hbm, srcf_hbm, wf_hbm, nd_hbm, c_hbm,
                 hist_v, idx_v, w_v, acc_v, tmp_v, shared):
    cx = lax.axis_index("c")
    s = lax.axis_index("s")
    pltpu.sync_copy(idx_hbm.at[pl.ds((cx * NT + s) * EPT, EPT)],
                    idx_v.at[pl.ds(0, EPT)])

    @pl.loop(0, NP, step=16)
    def _zero(i):
        hist_v[pl.ds(i, 16)] = jnp.zeros((16,), jnp.float32)

    @pl.loop(0, EPT, step=16)
    def _count(e0):
        idx16 = idx_v[pl.ds(e0, 16)]
        cnt, last = plsc.scan_count(idx16)
        plsc.addupdate_scatter(hist_v, [idx16], cnt.astype(jnp.float32),
                               mask=last)

    pltpu.sync_copy(hist_v, shared.at[pl.ds(s * NP, NP)])
    plsc.subcore_barrier()

    base = s * STRIDE

    @pl.loop(0, STRIDE, step=16)
    def _zacc(i):
        acc_v[pl.ds(i, 16)] = jnp.zeros((16,), jnp.float32)

    @pl.loop(0, NT)
    def _sum(t):
        pltpu.sync_copy(shared.at[pl.ds(t * NP + base, STRIDE)], tmp_v)

        @pl.loop(0, STRIDE, step=16)
        def _add(i):
            acc_v[pl.ds(i, 16)] = acc_v[pl.ds(i, 16)] + tmp_v[pl.ds(i, 16)]

    @pl.loop(0, STRIDE, step=16)
    def _norm(i):
        acc_v[pl.ds(i, 16)] = _rsqrt16(acc_v[pl.ds(i, 16)])

    plsc.subcore_barrier()

    @pl.when(cx == 1)
    def _():
        pltpu.sync_copy(acc_v, nd_hbm.at[pl.ds(base, STRIDE)])

    @pl.when(cx == 0)
    def _():
        pltpu.sync_copy(acc_v, shared.at[pl.ds(base, STRIDE)])

    plsc.subcore_barrier()

    @pl.when(cx == 0)
    def _():
        pltpu.sync_copy(shared.at[pl.ds(0, NP)], hist_v)
        pltpu.sync_copy(srcf_hbm.at[pl.ds(s * ECT, ECT)], idx_v)
        pltpu.sync_copy(wf_hbm.at[pl.ds(s * ECT, ECT)], w_v)

        @pl.loop(0, ECT, step=16)
        def _coef(e0):
            s16 = idx_v[pl.ds(e0, 16)]
            ns16 = plsc.load_gather(hist_v, [s16])
            w_v[pl.ds(e0, 16)] = w_v[pl.ds(e0, 16)] * ns16

        pltpu.sync_copy(w_v, c_hbm.at[pl.ds(s * ECT, ECT)])


@functools.partial(
    pl.kernel,
    out_type=jax.ShapeDtypeStruct((2 * NP, H), jnp.float32),
    mesh=_mesh,
    scratch_types=[
        pltpu.VMEM((NCH, CH), jnp.int32),
        pltpu.VMEM((NBUF, CH), jnp.int32),
        pltpu.VMEM((NBUF, CH), jnp.float32),
        [pltpu.VMEM((CH, H), jnp.float32)] * NBUF,
        pltpu.VMEM_SHARED((NP, H), jnp.float32),
        [pltpu.SemaphoreType.DMA] * NBUF,
        [pltpu.SemaphoreType.DMA] * NBUF,
    ],
    compiler_params=_sc_params,
)
def _spmm_kernel(x_hbm, src_hbm, dst_hbm, w_hbm, zero_hbm, out_hbm,
                 src_v, dst_v, w_v, rows, acc, gsem, ssem):
    c = lax.axis_index("c")
    s = lax.axis_index("s")
    pltpu.sync_copy(src_hbm.at[c * NT + s], src_v)

    def gstart(b, q):
        pltpu.async_copy(x_hbm.at[src_v.at[q]], rows[b], gsem[b])
        off = (s * NCH + q) * CH
        pltpu.async_copy(dst_hbm.at[pl.ds(off, CH)], dst_v.at[b], gsem[b])
        pltpu.async_copy(w_hbm.at[pl.ds(off, CH)], w_v.at[b], gsem[b])

    def gwait(b, q):
        pltpu.make_async_copy(x_hbm.at[src_v.at[q]], rows[b], gsem[b]).wait()
        off = (s * NCH + q) * CH
        pltpu.make_async_copy(dst_hbm.at[pl.ds(off, CH)], dst_v.at[b],
                              gsem[b]).wait()
        pltpu.make_async_copy(w_hbm.at[pl.ds(off, CH)], w_v.at[b],
                              gsem[b]).wait()

    def sstart(b):
        pltpu.async_copy(rows[b], acc.at[dst_v.at[b]], ssem[b], add=True)

    def swait(b):
        pltpu.make_async_copy(rows[b], acc.at[dst_v.at[b]], ssem[b]).wait()

    for b in range(NBUF - 1):
        gstart(b, b)

    pltpu.sync_copy(zero_hbm.at[pl.ds(s * STRIDE, STRIDE)],
                    acc.at[pl.ds(s * STRIDE, STRIDE)])
    plsc.subcore_barrier()

    @pl.loop(0, NCH, step=NBUF)
    def _iter(j):
        for k in range(NBUF):
            q = j + k
            b = k
            gwait(b, q)

            @pl.loop(0, CH, step=16)
            def _scale(r0):
                wv = w_v[b, pl.ds(r0, 16)]
                for r in range(16):
                    wr = wv[r]
                    for kk in range(H // 16):
                        sl = (r0 + r, pl.ds(kk * 16, 16))
                        rows[b][sl] = rows[b][sl] * wr

            sstart(b)
            nb = (k + NBUF - 1) % NBUF
            if k == 0:
                @pl.when(j > 0)
                def _():
                    swait(nb)

                gstart(nb, q + NBUF - 1)
            else:
                @pl.when(j < NCH - (NBUF - 1) - k)
                def _():
                    swait(nb)
                    gstart(nb, q + NBUF - 1)

    for b in range(NBUF):
        swait(b)
    plsc.subcore_barrier()

    pltpu.sync_copy(acc.at[pl.ds(s * STRIDE, STRIDE)],
                    out_hbm.at[pl.ds(c * NP + s * STRIDE, STRIDE)])


BN = 1280
BNF = 1000


def _mm1_body(x_ref, w_ref, o_ref):
    x = x_ref[...]
    w = w_ref[...]
    o_ref[0] = jnp.dot(x, w[:, :H], preferred_element_type=jnp.float32)
    o_ref[1] = jnp.dot(x, w[:, H:], preferred_element_type=jnp.float32)


def _mm2_body(a_ref, nd_ref, b_ref, w_ref, o_ref):
    nd = nd_ref[...]
    b = b_ref[...]
    ha = jnp.maximum(a_ref[0] * nd + b[0, :H], 0.0)
    hb = jnp.maximum(a_ref[1] * nd + b[0, H:], 0.0)
    w = w_ref[...]
    o_ref[0] = (jnp.dot(ha, w[:H, :H], preferred_element_type=jnp.float32)
                + jnp.dot(hb, w[H:, :H], preferred_element_type=jnp.float32))
    o_ref[1] = (jnp.dot(ha, w[:H, H:], preferred_element_type=jnp.float32)
                + jnp.dot(hb, w[H:, H:], preferred_element_type=jnp.float32))


def _fin_body(a_ref, nd_ref, b_ref, o_ref):
    nd = nd_ref[...]
    b = b_ref[...]
    o_ref[:, :H] = a_ref[0] * nd + b[0, :H]
    o_ref[:, H:] = a_ref[1] * nd + b[0, H:]


_row_spec = pl.BlockSpec((BN, D), lambda i: (i, 0))
_halves_spec = pl.BlockSpec((2, BN, H), lambda i: (0, i, 0))
_nd_spec = pl.BlockSpec((BN, 1), lambda i: (i, 0))
_w_spec = pl.BlockSpec((D, D), lambda i: (0, 0))
_b_spec = pl.BlockSpec((1, D), lambda i: (0, 0))

_mm1 = pl.pallas_call(
    _mm1_body,
    grid=(NP // BN,),
    in_specs=[_row_spec, _w_spec],
    out_specs=_halves_spec,
    out_shape=jax.ShapeDtypeStruct((2, NP, H), jnp.float32),
)

_mm2 = pl.pallas_call(
    _mm2_body,
    grid=(NP // BN,),
    in_specs=[_halves_spec, _nd_spec, _b_spec, _w_spec],
    out_specs=_halves_spec,
    out_shape=jax.ShapeDtypeStruct((2, NP, H), jnp.float32),
)

_fin = pl.pallas_call(
    _fin_body,
    grid=(N // BNF,),
    in_specs=[pl.BlockSpec((2, BNF, H), lambda i: (0, i, 0)),
              pl.BlockSpec((BNF, 1), lambda i: (i, 0)),
              _b_spec],
    out_specs=pl.BlockSpec((BNF, D), lambda i: (i, 0)),
    out_shape=jax.ShapeDtypeStruct((N, D), jnp.float32),
)


def kernel(features, edge_index, edge_weight, W1, b1, W2, b2):
    src = edge_index[0]
    dst = edge_index[1]

    idx_flat = jnp.concatenate([src, dst])
    pad = EP - E
    pad_idx = (jnp.arange(pad, dtype=jnp.int32) * 37) % N
    srcf = jnp.concatenate([src, pad_idx])
    srcp = srcf.reshape(NT, NCH, CH)
    srcp2 = jnp.concatenate([srcp, srcp + NP], axis=0)
    dstp = jnp.concatenate([dst, pad_idx])
    wp = jnp.concatenate([edge_weight, jnp.zeros((pad,), jnp.float32)])
    xpad = jnp.pad(features, ((0, NP - N), (0, 0)))
    zeros_half = jnp.zeros((NP, H), jnp.float32)
    b1r = b1.reshape(1, D)
    b2r = b2.reshape(1, D)

    nd, cw = _prep_kernel(idx_flat, srcf, wp)
    ndr = nd.reshape(NP, 1)
    x1 = _mm1(xpad, W1)
    a1 = _spmm_kernel(x1.reshape(2 * NP, H), srcp2, dstp, cw, zeros_half)
    x2 = _mm2(a1.reshape(2, NP, H), ndr, b1r, W2)
    a2 = _spmm_kernel(x2.reshape(2 * NP, H), srcp2, dstp, cw, zeros_half)
    return _fin(a2.reshape(2, NP, H), ndr[:N], b2r)


# speedup vs baseline: 7.0216x; 1.0085x over previous
import functools

import jax
import jax.numpy as jnp
from jax import lax
from jax.experimental import pallas as pl
from jax.experimental.pallas import tpu as pltpu
from jax.experimental.pallas import tpu_sc as plsc

N = 10000
E = 160000
D = 256
H = 128
NT = 16
NP = 10240
STRIDE = NP // NT
EPT = E // NT
CH = 64
NCH = 159
ECT = NCH * CH
EP = NT * ECT
NBUF = 3

_mesh = plsc.VectorSubcoreMesh(core_axis_name="c", subcore_axis_name="s")

_sc_params = pltpu.CompilerParams(needs_layout_passes=False)


def _rsqrt16(x):
    x = jnp.maximum(x, 1.0)
    i = plsc.bitcast(x, jnp.int32)
    i = 0x5F3759DF - lax.shift_right_logical(i, 1)
    y = plsc.bitcast(i, jnp.float32)
    for _ in range(3):
        y = y * (1.5 - 0.5 * x * y * y)
    return y


@functools.partial(
    pl.kernel,
    out_type=[
        jax.ShapeDtypeStruct((NP,), jnp.float32),
        jax.ShapeDtypeStruct((EP,), jnp.float32),
    ],
    mesh=_mesh,
    scratch_types=[
        pltpu.VMEM((NP,), jnp.float32),
        pltpu.VMEM((ECT,), jnp.int32),
        pltpu.VMEM((ECT,), jnp.float32),
        pltpu.VMEM((STRIDE,), jnp.float32),
        pltpu.VMEM((STRIDE,), jnp.float32),
        pltpu.VMEM_SHARED((NT * NP,), jnp.float32),
    ],
    compiler_params=_sc_params,
)
def _prep_kernel(idx_hbm, srcf_hbm, wf_hbm, nd_hbm, c_hbm,
                 hist_v, idx_v, w_v, acc_v, tmp_v, shared):
    cx = lax.axis_index("c")
    s = lax.axis_index("s")
    pltpu.sync_copy(idx_hbm.at[pl.ds((cx * NT + s) * EPT, EPT)],
                    idx_v.at[pl.ds(0, EPT)])

    @pl.loop(0, NP, step=16)
    def _zero(i):
        hist_v[pl.ds(i, 16)] = jnp.zeros((16,), jnp.float32)

    @pl.loop(0, EPT, step=16)
    def _count(e0):
        idx16 = idx_v[pl.ds(e0, 16)]
        cnt, last = plsc.scan_count(idx16)
        plsc.addupdate_scatter(hist_v, [idx16], cnt.astype(jnp.float32),
                               mask=last)

    pltpu.sync_copy(hist_v, shared.at[pl.ds(s * NP, NP)])
    plsc.subcore_barrier()

    base = s * STRIDE

    @pl.loop(0, STRIDE, step=16)
    def _zacc(i):
        acc_v[pl.ds(i, 16)] = jnp.zeros((16,), jnp.float32)

    @pl.loop(0, NT)
    def _sum(t):
        pltpu.sync_copy(shared.at[pl.ds(t * NP + base, STRIDE)], tmp_v)

        @pl.loop(0, STRIDE, step=16)
        def _add(i):
            acc_v[pl.ds(i, 16)] = acc_v[pl.ds(i, 16)] + tmp_v[pl.ds(i, 16)]

    @pl.loop(0, STRIDE, step=16)
    def _norm(i):
        acc_v[pl.ds(i, 16)] = _rsqrt16(acc_v[pl.ds(i, 16)])

    plsc.subcore_barrier()

    @pl.when(cx == 1)
    def _():
        pltpu.sync_copy(acc_v, nd_hbm.at[pl.ds(base, STRIDE)])

    @pl.when(cx == 0)
    def _():
        pltpu.sync_copy(acc_v, shared.at[pl.ds(base, STRIDE)])

    plsc.subcore_barrier()

    @pl.when(cx == 0)
    def _():
        pltpu.sync_copy(shared.at[pl.ds(0, NP)], hist_v)
        pltpu.sync_copy(srcf_hbm.at[pl.ds(s * ECT, ECT)], idx_v)
        pltpu.sync_copy(wf_hbm.at[pl.ds(s * ECT, ECT)], w_v)

        @pl.loop(0, ECT, step=16)
        def _coef(e0):
            s16 = idx_v[pl.ds(e0, 16)]
            ns16 = plsc.load_gather(hist_v, [s16])
            w_v[pl.ds(e0, 16)] = w_v[pl.ds(e0, 16)] * ns16

        pltpu.sync_copy(w_v, c_hbm.at[pl.ds(s * ECT, ECT)])


@functools.partial(
    pl.kernel,
    out_type=jax.ShapeDtypeStruct((2 * NP, H), jnp.float32),
    mesh=_mesh,
    scratch_types=[
        pltpu.VMEM((NCH, CH), jnp.int32),
        pltpu.VMEM((NBUF, CH), jnp.int32),
        pltpu.VMEM((NBUF, CH), jnp.float32),
        [pltpu.VMEM((CH, H), jnp.float32)] * NBUF,
        pltpu.VMEM_SHARED((NP, H), jnp.float32),
        [pltpu.SemaphoreType.DMA] * NBUF,
        [pltpu.SemaphoreType.DMA] * NBUF,
    ],
    compiler_params=_sc_params,
)
def _spmm_kernel(x_hbm, src_hbm, dst_hbm, w_hbm, zero_hbm, out_hbm,
                 src_v, dst_v, w_v, rows, acc, gsem, ssem):
    c = lax.axis_index("c")
    s = lax.axis_index("s")
    pltpu.sync_copy(src_hbm.at[c * NT + s], src_v)

    def gstart(b, q):
        pltpu.async_copy(x_hbm.at[src_v.at[q]], rows[b], gsem[b])
        off = (s * NCH + q) * CH
        pltpu.async_copy(dst_hbm.at[pl.ds(off, CH)], dst_v.at[b], gsem[b])
        pltpu.async_copy(w_hbm.at[pl.ds(off, CH)], w_v.at[b], gsem[b])

    def gwait(b, q):
        pltpu.make_async_copy(x_hbm.at[src_v.at[q]], rows[b], gsem[b]).wait()
        off = (s * NCH + q) * CH
        pltpu.make_async_copy(dst_hbm.at[pl.ds(off, CH)], dst_v.at[b],
                              gsem[b]).wait()
        pltpu.make_async_copy(w_hbm.at[pl.ds(off, CH)], w_v.at[b],
                              gsem[b]).wait()

    def sstart(b):
        pltpu.async_copy(rows[b], acc.at[dst_v.at[b]], ssem[b], add=True)

    def swait(b):
        pltpu.make_async_copy(rows[b], acc.at[dst_v.at[b]], ssem[b]).wait()

    for b in range(NBUF - 1):
        gstart(b, b)

    pltpu.sync_copy(zero_hbm.at[pl.ds(s * STRIDE, STRIDE)],
                    acc.at[pl.ds(s * STRIDE, STRIDE)])
    plsc.subcore_barrier()

    @pl.loop(0, NCH, step=NBUF)
    def _iter(j):
        for k in range(NBUF):
            q = j + k
            b = k
            gwait(b, q)

            @pl.loop(0, CH, step=16)
            def _scale(r0):
                wv = w_v[b, pl.ds(r0, 16)]
                for r in range(16):
                    wr = wv[r]
                    for kk in range(H // 16):
                        sl = (r0 + r, pl.ds(kk * 16, 16))
                        rows[b][sl] = rows[b][sl] * wr

            sstart(b)
            nb = (k + NBUF - 1) % NBUF
            if k == 0:
                @pl.when(j > 0)
                def _():
                    swait(nb)

                gstart(nb, q + NBUF - 1)
            else:
                @pl.when(j < NCH - (NBUF - 1) - k)
                def _():
                    swait(nb)
                    gstart(nb, q + NBUF - 1)

    for b in range(NBUF):
        swait(b)
    plsc.subcore_barrier()

    pltpu.sync_copy(acc.at[pl.ds(s * STRIDE, STRIDE)],
                    out_hbm.at[pl.ds(c * NP + s * STRIDE, STRIDE)])


BN = 1280
BNF = 1000


def _mm1_body(x_ref, w_ref, o_ref):
    x = x_ref[...]
    w = w_ref[...]
    o_ref[0] = jnp.dot(x, w[:, :H], preferred_element_type=jnp.float32)
    o_ref[1] = jnp.dot(x, w[:, H:], preferred_element_type=jnp.float32)


def _mm2_body(a_ref, nd_ref, b_ref, w_ref, o_ref):
    nd = nd_ref[...]
    b = b_ref[...]
    ha = jnp.maximum(a_ref[0] * nd + b[0, :H], 0.0)
    hb = jnp.maximum(a_ref[1] * nd + b[0, H:], 0.0)
    w = w_ref[...]
    o_ref[0] = (jnp.dot(ha, w[:H, :H], preferred_element_type=jnp.float32)
                + jnp.dot(hb, w[H:, :H], preferred_element_type=jnp.float32))
    o_ref[1] = (jnp.dot(ha, w[:H, H:], preferred_element_type=jnp.float32)
                + jnp.dot(hb, w[H:, H:], preferred_element_type=jnp.float32))


def _fin_body(a_ref, nd_ref, b_ref, o_ref):
    nd = nd_ref[...]
    b = b_ref[...]
    o_ref[:, :H] = a_ref[0] * nd + b[0, :H]
    o_ref[:, H:] = a_ref[1] * nd + b[0, H:]


_row_spec = pl.BlockSpec((BN, D), lambda i: (i, 0))
_halves_spec = pl.BlockSpec((2, BN, H), lambda i: (0, i, 0))
_nd_spec = pl.BlockSpec((BN, 1), lambda i: (i, 0))
_w_spec = pl.BlockSpec((D, D), lambda i: (0, 0))
_b_spec = pl.BlockSpec((1, D), lambda i: (0, 0))

_mm1 = pl.pallas_call(
    _mm1_body,
    grid=(NP // BN,),
    in_specs=[_row_spec, _w_spec],
    out_specs=_halves_spec,
    out_shape=jax.ShapeDtypeStruct((2, NP, H), jnp.float32),
)

_mm2 = pl.pallas_call(
    _mm2_body,
    grid=(NP // BN,),
    in_specs=[_halves_spec, _nd_spec, _b_spec, _w_spec],
    out_specs=_halves_spec,
    out_shape=jax.ShapeDtypeStruct((2, NP, H), jnp.float32),
)

_fin = pl.pallas_call(
    _fin_body,
    grid=(N // BNF,),
    in_specs=[pl.BlockSpec((2, BNF, H), lambda i: (0, i, 0)),
              pl.BlockSpec((BNF, 1), lambda i: (i, 0)),
              _b_spec],
    out_specs=pl.BlockSpec((BNF, D), lambda i: (i, 0)),
    out_shape=jax.ShapeDtypeStruct((N, D), jnp.float32),
)


def kernel(features, edge_index, edge_weight, W1, b1, W2, b2):
    src = edge_index[0]
    dst = edge_index[1]

    idx_flat = jnp.concatenate([src, dst])
    pad = EP - E
    pad_idx = (jnp.arange(pad, dtype=jnp.int32) * 37) % N
    srcf = jnp.concatenate([src, pad_idx])
    srcp = srcf.reshape(NT, NCH, CH)
    srcp2 = jnp.concatenate([srcp, srcp + NP], axis=0)
    dstp = jnp.concatenate([dst, pad_idx])
    wp = jnp.concatenate([edge_weight, jnp.zeros((pad,), jnp.float32)])
    xpad = jnp.pad(features, ((0, NP - N), (0, 0)))
    zeros_half = jnp.zeros((NP, H), jnp.float32)
    b1r = b1.reshape(1, D)
    b2r = b2.reshape(1, D)

    nd, cw = _prep_kernel(idx_flat, srcf, wp)
    ndr = nd.reshape(NP, 1)
    x1 = _mm1(xpad, W1)
    a1 = _spmm_kernel(x1.reshape(2 * NP, H), srcp2, dstp, cw, zeros_half)
    x2 = _mm2(a1.reshape(2, NP, H), ndr, b1r, W2)
    a2 = _spmm_kernel(x2.reshape(2 * NP, H), srcp2, dstp, cw, zeros_half)
    return _fin(a2.reshape(2, NP, H), ndr[:N], b2r)
